# baseline (speedup 1.0000x reference)
import functools

import jax
import jax.numpy as jnp
import numpy as np
from jax import lax
from jax.experimental import pallas as pl
from jax.experimental.pallas import tpu as pltpu
from jax.experimental.pallas import tpu_sc as plsc

_NC = 2
_NS = 16
_VB = 16384
_LOG2VB = 14
_CH = 128


def _transpose_body(tt_ref, out_ref):
    x = tt_ref[...]
    D = x.shape[0]
    h = D // 2
    y = x.astype(jnp.bfloat16).T.astype(jnp.float32)
    yi = lax.bitcast_convert_type(y, jnp.int32)
    q = _VB // 4
    for s in range(4):
        ys = yi[s * q:(s + 1) * q]
        w = lax.bitwise_or(ys[:, :h],
                           lax.shift_right_logical(ys[:, h:], 16))
        out_ref[:, s * h:(s + 1) * h] = lax.bitcast_convert_type(
            w, jnp.float32)


def _tc_transpose(tt):
    D, V = tt.shape
    nblk = (V + _VB - 1) // _VB
    return pl.pallas_call(
        _transpose_body,
        grid=(nblk,),
        in_specs=[pl.BlockSpec((D, _VB), lambda i: (0, i))],
        out_specs=pl.BlockSpec((_VB // 4, 2 * D), lambda i: (i, 0)),
        out_shape=jax.ShapeDtypeStruct((nblk * _VB // 4, 2 * D), jnp.float32),
    )(tt)


def _sc_gather(tr2, idx1, idx2, D):
    P = tr2.shape[0]
    B = idx1.shape[0]
    NW = _NC * _NS
    bpw = B // NW
    L = 16
    mesh = plsc.VectorSubcoreMesh(core_axis_name="c", subcore_axis_name="s")

    @functools.partial(
        pl.kernel,
        out_type=(
            jax.ShapeDtypeStruct((B, 2 * D), jnp.float32),
            jax.ShapeDtypeStruct((B, 2 * D), jnp.float32),
        ),
        mesh=mesh,
        compiler_params=pltpu.CompilerParams(needs_layout_passes=False),
        scratch_types=[
            pltpu.VMEM((bpw,), jnp.int32),
            pltpu.VMEM((_CH,), jnp.int32),
            pltpu.VMEM((_CH,), jnp.int32),
            pltpu.VMEM((_CH, 2 * D), jnp.float32),
            pltpu.VMEM((_CH, 2 * D), jnp.float32),
            pltpu.SemaphoreType.DMA,
            pltpu.SemaphoreType.DMA,
        ],
    )
    def gather_k(tr2_hbm, idx1_hbm, idx2_hbm, out1_hbm, out2_hbm,
                 idx_v, pidx0_v, pidx1_v, g0_v, g1_v, sem0, sem1):
        wid = lax.axis_index("s") * _NC + lax.axis_index("c")
        base = wid * bpw
        pidx = (pidx0_v, pidx1_v)
        gbuf = (g0_v, g1_v)
        sems = (sem0, sem1)
        nch = bpw // _CH

        def one_side(idx_hbm, out_hbm):
            pltpu.sync_copy(idx_hbm.at[pl.ds(base, bpw)], idx_v)

            def fire(ch):
                k = ch % 2

                def shift_body(g, _):
                    v16 = idx_v[pl.ds(ch * _CH + g * L, L)]
                    pidx[k][pl.ds(g * L, L)] = (
                        lax.shift_left(
                            lax.shift_right_logical(v16, _LOG2VB),
                            _LOG2VB - 2)
                        + lax.bitwise_and(v16, _VB // 4 - 1))
                    return 0

                lax.fori_loop(0, _CH // L, shift_body, 0)
                return pltpu.async_copy(tr2_hbm.at[pidx[k]], gbuf[k], sems[k])

            cp = fire(0)
            for ch in range(nch):
                if ch + 1 < nch:
                    cp_next = fire(ch + 1)
                cp.wait()
                pltpu.sync_copy(gbuf[ch % 2],
                                out_hbm.at[pl.ds(base + ch * _CH, _CH)])
                if ch + 1 < nch:
                    cp = cp_next

        one_side(idx1_hbm, out1_hbm)
        one_side(idx2_hbm, out2_hbm)

    return gather_k(tr2, idx1, idx2)


def _mlp_body(g1_ref, g2_ref, s1_ref, e1_ref, s2_ref, e2_ref, wp1_ref,
              wp2_ref, bp_ref, w1_ref, b1_ref, gam_ref, bet_ref, w2_ref,
              b2_ref, gm_ref, fire_ref, feats_ref):
    H2 = g1_ref.shape[1] // 2
    Q = H2 // 2

    def sel(g_ref, s_ref, e_ref):
        g = g_ref[...]
        a = jnp.where(e_ref[...] > 0.5, g[:, H2:], g[:, :H2])
        b = jnp.where(s_ref[...] > 0.5, a[:, Q:], a[:, :Q])
        bi = lax.bitcast_convert_type(b, jnp.int32)
        xhi = lax.bitcast_convert_type(
            lax.bitwise_and(bi, jnp.int32(-65536)), jnp.float32)
        xlo = lax.bitcast_convert_type(
            lax.shift_left(bi, 16), jnp.float32)
        return xhi, xlo

    xh1, xl1 = sel(g1_ref, s1_ref, e1_ref)
    xh2, xl2 = sel(g2_ref, s2_ref, e2_ref)
    wp1 = wp1_ref[...]
    wp2 = wp2_ref[...]
    z = (jnp.dot(xh1, wp1[:Q], preferred_element_type=jnp.float32)
         + jnp.dot(xl1, wp1[Q:], preferred_element_type=jnp.float32)
         + jnp.dot(xh2, wp2[:Q], preferred_element_type=jnp.float32)
         + jnp.dot(xl2, wp2[Q:], preferred_element_type=jnp.float32)
         + bp_ref[...])
    feats = jax.nn.sigmoid(z)
    feats_ref[...] = feats
    nin = 1.5 * feats - 0.5 * gm_ref[...]
    h = jnp.dot(nin, w1_ref[...], preferred_element_type=jnp.float32) + b1_ref[...]
    mu = jnp.mean(h, axis=-1, keepdims=True)
    var = jnp.mean((h - mu) ** 2, axis=-1, keepdims=True)
    hn = (h - mu) / jnp.sqrt(var + 1e-5) * gam_ref[...] + bet_ref[...]
    hg = 0.5 * hn * (1.0 + lax.erf(hn * np.float32(1.0 / np.sqrt(2.0))))
    pot = jnp.dot(hg, w2_ref[...], preferred_element_type=jnp.float32) + b2_ref[...]
    fire_ref[...] = jax.nn.sigmoid(pot)


def _tc_mlp(g1, g2, s1, e1, s2, e2, Wp, bp, W1, b1, gamma, beta, W2, b2,
            global_mean):
    B = g1.shape[0]
    D = 64
    BLK = 2048
    wp1t = Wp[:, :D].T
    wp2t = Wp[:, D:].T
    w1t = W1.T
    w2t = W2.T
    row = lambda v: v.reshape(1, -1)
    full = lambda a: pl.BlockSpec(a.shape, lambda i: (0,) * a.ndim)
    return pl.pallas_call(
        _mlp_body,
        grid=(B // BLK,),
        in_specs=[
            pl.BlockSpec((BLK, 2 * D), lambda i: (i, 0)),
            pl.BlockSpec((BLK, 2 * D), lambda i: (i, 0)),
            pl.BlockSpec((BLK, 1), lambda i: (i, 0)),
            pl.BlockSpec((BLK, 1), lambda i: (i, 0)),
            pl.BlockSpec((BLK, 1), lambda i: (i, 0)),
            pl.BlockSpec((BLK, 1), lambda i: (i, 0)),
            full(wp1t), full(wp2t), full(row(bp)),
            full(w1t), full(row(b1)), full(row(gamma)), full(row(beta)),
            full(w2t), full(row(b2)), full(row(global_mean)),
        ],
        out_specs=[
            pl.BlockSpec((BLK, 1), lambda i: (i, 0)),
            pl.BlockSpec((BLK, 4), lambda i: (i, 0)),
        ],
        out_shape=[
            jax.ShapeDtypeStruct((B, 1), jnp.float32),
            jax.ShapeDtypeStruct((B, 4), jnp.float32),
        ],
    )(g1, g2, s1, e1, s2, e2, wp1t, wp2t, row(bp), w1t, row(b1), row(gamma),
      row(beta), w2t, row(b2), row(global_mean))


def kernel(idx1, idx2, table, Wp, bp, W1, b1, gamma, beta, W2, b2, global_mean):
    D = table.shape[1]
    i1 = idx1.astype(jnp.int32)
    i2 = idx2.astype(jnp.int32)
    tr2 = _tc_transpose(table.T)
    g1, g2 = _sc_gather(tr2, i1, i2, D)
    bit = lambda ix, k: jnp.reshape(
        lax.bitwise_and(lax.shift_right_logical(ix, k), 1).astype(jnp.float32),
        (-1, 1))
    firing, feats = _tc_mlp(g1, g2, bit(i1, _LOG2VB - 2), bit(i1, _LOG2VB - 1),
                            bit(i2, _LOG2VB - 2), bit(i2, _LOG2VB - 1),
                            Wp, bp, W1, b1, gamma, beta, W2, b2, global_mean)
    return firing, feats


# speedup vs baseline: 1.3633x; 1.3633x over previous
import functools

import jax
import jax.numpy as jnp
import numpy as np
from jax import lax
from jax.experimental import pallas as pl
from jax.experimental.pallas import tpu as pltpu
from jax.experimental.pallas import tpu_sc as plsc

_NC = 2
_NS = 16
_VB = 16384
_LOG2VB = 14
_CH = 128


def _transpose_body(tt_ref, out_ref):
    x = tt_ref[...]
    D = x.shape[0]
    y = x.T
    h = _VB // 2
    out_ref[:, :D] = y[:h]
    out_ref[:, D:] = y[h:]


def _tc_transpose(tt):
    D, V = tt.shape
    nblk = (V + _VB - 1) // _VB
    return pl.pallas_call(
        _transpose_body,
        grid=(nblk,),
        in_specs=[pl.BlockSpec((D, _VB), lambda i: (0, i))],
        out_specs=pl.BlockSpec((_VB // 2, 2 * D), lambda i: (i, 0)),
        out_shape=jax.ShapeDtypeStruct((nblk * _VB // 2, 2 * D), jnp.float32),
    )(tt)


def _sc_gather(tr2, idx1, idx2, D):
    P = tr2.shape[0]
    B = idx1.shape[0]
    NW = _NC * _NS
    bpw = B // NW
    L = 16
    mesh = plsc.VectorSubcoreMesh(core_axis_name="c", subcore_axis_name="s")

    @functools.partial(
        pl.kernel,
        out_type=(
            jax.ShapeDtypeStruct((B, 2 * D), jnp.float32),
            jax.ShapeDtypeStruct((B, 2 * D), jnp.float32),
        ),
        mesh=mesh,
        compiler_params=pltpu.CompilerParams(needs_layout_passes=False),
        scratch_types=[
            pltpu.VMEM((bpw,), jnp.int32),
            pltpu.VMEM((_CH,), jnp.int32),
            pltpu.VMEM((_CH,), jnp.int32),
            pltpu.VMEM((_CH, 2 * D), jnp.float32),
            pltpu.VMEM((_CH, 2 * D), jnp.float32),
            pltpu.SemaphoreType.DMA,
            pltpu.SemaphoreType.DMA,
        ],
    )
    def gather_k(tr2_hbm, idx1_hbm, idx2_hbm, out1_hbm, out2_hbm,
                 idx_v, pidx0_v, pidx1_v, g0_v, g1_v, sem0, sem1):
        wid = lax.axis_index("s") * _NC + lax.axis_index("c")
        base = wid * bpw
        pidx = (pidx0_v, pidx1_v)
        gbuf = (g0_v, g1_v)
        sems = (sem0, sem1)
        nch = bpw // _CH

        def one_side(idx_hbm, out_hbm):
            pltpu.sync_copy(idx_hbm.at[pl.ds(base, bpw)], idx_v)

            def fire(ch):
                k = ch % 2

                def shift_body(g, _):
                    v16 = idx_v[pl.ds(ch * _CH + g * L, L)]
                    pidx[k][pl.ds(g * L, L)] = (
                        lax.shift_left(
                            lax.shift_right_logical(v16, _LOG2VB),
                            _LOG2VB - 1)
                        + lax.bitwise_and(v16, _VB // 2 - 1))
                    return 0

                lax.fori_loop(0, _CH // L, shift_body, 0)
                return pltpu.async_copy(tr2_hbm.at[pidx[k]], gbuf[k], sems[k])

            cp = fire(0)
            for ch in range(nch):
                if ch + 1 < nch:
                    cp_next = fire(ch + 1)
                cp.wait()
                pltpu.sync_copy(gbuf[ch % 2],
                                out_hbm.at[pl.ds(base + ch * _CH, _CH)])
                if ch + 1 < nch:
                    cp = cp_next

        one_side(idx1_hbm, out1_hbm)
        one_side(idx2_hbm, out2_hbm)

    return gather_k(tr2, idx1, idx2)


def _mlp_body(g1_ref, g2_ref, m1_ref, m2_ref, wp1_ref, wp2_ref, bp_ref,
              w1_ref, b1_ref, gam_ref, bet_ref, w2_ref, b2_ref, gm_ref,
              fire_ref, feats_ref):
    D = g1_ref.shape[1] // 2
    x1 = jnp.where(m1_ref[...] > 0.5, g1_ref[:, D:], g1_ref[:, :D])
    x2 = jnp.where(m2_ref[...] > 0.5, g2_ref[:, D:], g2_ref[:, :D])
    z = (jnp.dot(x1, wp1_ref[...], preferred_element_type=jnp.float32)
         + jnp.dot(x2, wp2_ref[...], preferred_element_type=jnp.float32)
         + bp_ref[...])
    feats = jax.nn.sigmoid(z)
    feats_ref[...] = feats
    nin = 1.5 * feats - 0.5 * gm_ref[...]
    h = jnp.dot(nin, w1_ref[...], preferred_element_type=jnp.float32) + b1_ref[...]
    mu = jnp.mean(h, axis=-1, keepdims=True)
    var = jnp.mean((h - mu) ** 2, axis=-1, keepdims=True)
    hn = (h - mu) / jnp.sqrt(var + 1e-5) * gam_ref[...] + bet_ref[...]
    hg = 0.5 * hn * (1.0 + lax.erf(hn * np.float32(1.0 / np.sqrt(2.0))))
    pot = jnp.dot(hg, w2_ref[...], preferred_element_type=jnp.float32) + b2_ref[...]
    fire_ref[...] = jax.nn.sigmoid(pot)


def _tc_mlp(g1, g2, m1, m2, Wp, bp, W1, b1, gamma, beta, W2, b2, global_mean):
    B = g1.shape[0]
    D = g1.shape[1] // 2
    BLK = 2048
    wp1t = Wp[:, :D].T
    wp2t = Wp[:, D:].T
    w1t = W1.T
    w2t = W2.T
    row = lambda v: v.reshape(1, -1)
    full = lambda a: pl.BlockSpec(a.shape, lambda i: (0,) * a.ndim)
    return pl.pallas_call(
        _mlp_body,
        grid=(B // BLK,),
        in_specs=[
            pl.BlockSpec((BLK, 2 * D), lambda i: (i, 0)),
            pl.BlockSpec((BLK, 2 * D), lambda i: (i, 0)),
            pl.BlockSpec((BLK, 1), lambda i: (i, 0)),
            pl.BlockSpec((BLK, 1), lambda i: (i, 0)),
            full(wp1t), full(wp2t), full(row(bp)),
            full(w1t), full(row(b1)), full(row(gamma)), full(row(beta)),
            full(w2t), full(row(b2)), full(row(global_mean)),
        ],
        out_specs=[
            pl.BlockSpec((BLK, 1), lambda i: (i, 0)),
            pl.BlockSpec((BLK, 4), lambda i: (i, 0)),
        ],
        out_shape=[
            jax.ShapeDtypeStruct((B, 1), jnp.float32),
            jax.ShapeDtypeStruct((B, 4), jnp.float32),
        ],
    )(g1, g2, m1, m2, wp1t, wp2t, row(bp), w1t, row(b1), row(gamma),
      row(beta), w2t, row(b2), row(global_mean))


def kernel(idx1, idx2, table, Wp, bp, W1, b1, gamma, beta, W2, b2, global_mean):
    D = table.shape[1]
    i1 = idx1.astype(jnp.int32)
    i2 = idx2.astype(jnp.int32)
    tr2 = _tc_transpose(table.T)
    g1, g2 = _sc_gather(tr2, i1, i2, D)
    half = lambda ix: jnp.reshape(
        lax.bitwise_and(lax.shift_right_logical(ix, _LOG2VB - 1),
                        1).astype(jnp.float32), (-1, 1))
    firing, feats = _tc_mlp(g1, g2, half(i1), half(i2), Wp, bp, W1, b1,
                            gamma, beta, W2, b2, global_mean)
    return firing, feats


# speedup vs baseline: 1.3641x; 1.0006x over previous
import functools

import jax
import jax.numpy as jnp
import numpy as np
from jax import lax
from jax.experimental import pallas as pl
from jax.experimental.pallas import tpu as pltpu
from jax.experimental.pallas import tpu_sc as plsc

_NC = 2
_NS = 16
_VB = 16384
_LOG2VB = 14
_CH = 256


def _transpose_body(tt_ref, out_ref):
    x = tt_ref[...]
    D = x.shape[0]
    y = x.T
    h = _VB // 2
    out_ref[:, :D] = y[:h]
    out_ref[:, D:] = y[h:]


def _tc_transpose(tt):
    D, V = tt.shape
    nblk = (V + _VB - 1) // _VB
    return pl.pallas_call(
        _transpose_body,
        grid=(nblk,),
        in_specs=[pl.BlockSpec((D, _VB), lambda i: (0, i))],
        out_specs=pl.BlockSpec((_VB // 2, 2 * D), lambda i: (i, 0)),
        out_shape=jax.ShapeDtypeStruct((nblk * _VB // 2, 2 * D), jnp.float32),
    )(tt)


def _sc_gather(tr2, idx1, idx2, D):
    P = tr2.shape[0]
    B = idx1.shape[0]
    NW = _NC * _NS
    bpw = B // NW
    L = 16
    mesh = plsc.VectorSubcoreMesh(core_axis_name="c", subcore_axis_name="s")

    @functools.partial(
        pl.kernel,
        out_type=(
            jax.ShapeDtypeStruct((B, 2 * D), jnp.float32),
            jax.ShapeDtypeStruct((B, 2 * D), jnp.float32),
        ),
        mesh=mesh,
        compiler_params=pltpu.CompilerParams(needs_layout_passes=False),
        scratch_types=[
            pltpu.VMEM((bpw,), jnp.int32),
            pltpu.VMEM((_CH,), jnp.int32),
            pltpu.VMEM((_CH,), jnp.int32),
            pltpu.VMEM((_CH, 2 * D), jnp.float32),
            pltpu.VMEM((_CH, 2 * D), jnp.float32),
            pltpu.SemaphoreType.DMA,
            pltpu.SemaphoreType.DMA,
        ],
    )
    def gather_k(tr2_hbm, idx1_hbm, idx2_hbm, out1_hbm, out2_hbm,
                 idx_v, pidx0_v, pidx1_v, g0_v, g1_v, sem0, sem1):
        wid = lax.axis_index("s") * _NC + lax.axis_index("c")
        base = wid * bpw
        pidx = (pidx0_v, pidx1_v)
        gbuf = (g0_v, g1_v)
        sems = (sem0, sem1)
        nch = bpw // _CH

        def one_side(idx_hbm, out_hbm):
            pltpu.sync_copy(idx_hbm.at[pl.ds(base, bpw)], idx_v)

            def fire(ch):
                k = ch % 2

                def shift_body(g, _):
                    v16 = idx_v[pl.ds(ch * _CH + g * L, L)]
                    pidx[k][pl.ds(g * L, L)] = (
                        lax.shift_left(
                            lax.shift_right_logical(v16, _LOG2VB),
                            _LOG2VB - 1)
                        + lax.bitwise_and(v16, _VB // 2 - 1))
                    return 0

                lax.fori_loop(0, _CH // L, shift_body, 0)
                return pltpu.async_copy(tr2_hbm.at[pidx[k]], gbuf[k], sems[k])

            cp = fire(0)
            for ch in range(nch):
                if ch + 1 < nch:
                    cp_next = fire(ch + 1)
                cp.wait()
                pltpu.sync_copy(gbuf[ch % 2],
                                out_hbm.at[pl.ds(base + ch * _CH, _CH)])
                if ch + 1 < nch:
                    cp = cp_next

        one_side(idx1_hbm, out1_hbm)
        one_side(idx2_hbm, out2_hbm)

    return gather_k(tr2, idx1, idx2)


def _mlp_body(g1_ref, g2_ref, m1_ref, m2_ref, wp1_ref, wp2_ref, bp_ref,
              w1_ref, b1_ref, gam_ref, bet_ref, w2_ref, b2_ref, gm_ref,
              fire_ref, feats_ref):
    D = g1_ref.shape[1] // 2
    x1 = jnp.where(m1_ref[...] > 0.5, g1_ref[:, D:], g1_ref[:, :D])
    x2 = jnp.where(m2_ref[...] > 0.5, g2_ref[:, D:], g2_ref[:, :D])
    z = (jnp.dot(x1, wp1_ref[...], preferred_element_type=jnp.float32)
         + jnp.dot(x2, wp2_ref[...], preferred_element_type=jnp.float32)
         + bp_ref[...])
    feats = jax.nn.sigmoid(z)
    feats_ref[...] = feats
    nin = 1.5 * feats - 0.5 * gm_ref[...]
    h = jnp.dot(nin, w1_ref[...], preferred_element_type=jnp.float32) + b1_ref[...]
    mu = jnp.mean(h, axis=-1, keepdims=True)
    var = jnp.mean((h - mu) ** 2, axis=-1, keepdims=True)
    hn = (h - mu) / jnp.sqrt(var + 1e-5) * gam_ref[...] + bet_ref[...]
    hg = 0.5 * hn * (1.0 + lax.erf(hn * np.float32(1.0 / np.sqrt(2.0))))
    pot = jnp.dot(hg, w2_ref[...], preferred_element_type=jnp.float32) + b2_ref[...]
    fire_ref[...] = jax.nn.sigmoid(pot)


def _tc_mlp(g1, g2, m1, m2, Wp, bp, W1, b1, gamma, beta, W2, b2, global_mean):
    B = g1.shape[0]
    D = g1.shape[1] // 2
    BLK = 2048
    wp1t = Wp[:, :D].T
    wp2t = Wp[:, D:].T
    w1t = W1.T
    w2t = W2.T
    row = lambda v: v.reshape(1, -1)
    full = lambda a: pl.BlockSpec(a.shape, lambda i: (0,) * a.ndim)
    return pl.pallas_call(
        _mlp_body,
        grid=(B // BLK,),
        in_specs=[
            pl.BlockSpec((BLK, 2 * D), lambda i: (i, 0)),
            pl.BlockSpec((BLK, 2 * D), lambda i: (i, 0)),
            pl.BlockSpec((BLK, 1), lambda i: (i, 0)),
            pl.BlockSpec((BLK, 1), lambda i: (i, 0)),
            full(wp1t), full(wp2t), full(row(bp)),
            full(w1t), full(row(b1)), full(row(gamma)), full(row(beta)),
            full(w2t), full(row(b2)), full(row(global_mean)),
        ],
        out_specs=[
            pl.BlockSpec((BLK, 1), lambda i: (i, 0)),
            pl.BlockSpec((BLK, 4), lambda i: (i, 0)),
        ],
        out_shape=[
            jax.ShapeDtypeStruct((B, 1), jnp.float32),
            jax.ShapeDtypeStruct((B, 4), jnp.float32),
        ],
    )(g1, g2, m1, m2, wp1t, wp2t, row(bp), w1t, row(b1), row(gamma),
      row(beta), w2t, row(b2), row(global_mean))


def kernel(idx1, idx2, table, Wp, bp, W1, b1, gamma, beta, W2, b2, global_mean):
    D = table.shape[1]
    i1 = idx1.astype(jnp.int32)
    i2 = idx2.astype(jnp.int32)
    tr2 = _tc_transpose(table.T)
    g1, g2 = _sc_gather(tr2, i1, i2, D)
    half = lambda ix: jnp.reshape(
        lax.bitwise_and(lax.shift_right_logical(ix, _LOG2VB - 1),
                        1).astype(jnp.float32), (-1, 1))
    firing, feats = _tc_mlp(g1, g2, half(i1), half(i2), Wp, bp, W1, b1,
                            gamma, beta, W2, b2, global_mean)
    return firing, feats
